# baseline (speedup 1.0000x reference)
import functools

import jax
import jax.numpy as jnp
from jax import lax
from jax.experimental import pallas as pl
from jax.experimental.pallas import tpu as pltpu
from jax.experimental.pallas import tpu_sc as plsc

_N = 10000
_E = 320000
_NC, _NS = 2, 16
_NW = _NC * _NS
_EPT = _E // _NW
_B = 100
_NBLK = _EPT // _B
_PROP_B = {128: 100, 48: 100}
_PROP_RING = {128: 2, 48: 5}
_NP = 10240
_RPT = _NP // _NS

_ROWBLK = 1000
_G = _N // _ROWBLK

@functools.cache
def _get_mesh():
  return plsc.VectorSubcoreMesh(core_axis_name="c", subcore_axis_name="s")


_SC_PARAMS = pltpu.CompilerParams(use_tc_tiling_on_sc=False)


@functools.cache
def _make_prop(F):
  B = _PROP_B[F]
  NBLK = _EPT // B
  RING = _PROP_RING[F]

  @functools.partial(
      pl.kernel,
      out_type=jax.ShapeDtypeStruct((_NC, _NP, F), jnp.float32),
      mesh=_get_mesh(),
      compiler_params=_SC_PARAMS,
      scratch_types=[
          pltpu.VMEM((NBLK, B), jnp.int32),
          pltpu.VMEM((NBLK, B), jnp.int32),
          pltpu.VMEM((RING, B, F), jnp.float32),
          pltpu.VMEM_SHARED((_NP, F), jnp.float32),
          pltpu.SemaphoreType.DMA((RING,)),
          pltpu.SemaphoreType.DMA((RING,)),
      ],
  )
  def prop(h_hbm, e_hbm, z_hbm, out_hbm, src_v, dst_v, rows_v, acc_sh,
           gsem, ssem):
    cid = lax.axis_index("c")
    sid = lax.axis_index("s")
    wid = cid * _NS + sid
    pltpu.sync_copy(e_hbm.at[0, wid], src_v)
    pltpu.sync_copy(e_hbm.at[1, wid], dst_v)
    pltpu.async_copy(h_hbm.at[src_v.at[0]], rows_v.at[0], gsem.at[0])
    pltpu.async_copy(h_hbm.at[src_v.at[1]], rows_v.at[1], gsem.at[1])
    pltpu.sync_copy(z_hbm, acc_sh.at[pl.ds(sid * _RPT, _RPT)])
    plsc.subcore_barrier()

    @pl.loop(0, NBLK // RING)
    def _(i):
      for j in range(RING):
        b = RING * i + j
        k = (j + 2) % RING
        pltpu.make_async_copy(h_hbm.at[src_v.at[b]], rows_v.at[j],
                              gsem.at[j]).wait()
        pltpu.async_copy(rows_v.at[j], acc_sh.at[dst_v.at[b]], ssem.at[j],
                         add=True)

        @pl.when(b + 2 < NBLK)
        def _():
          @pl.when(b >= RING - 2)
          def _():
            pltpu.make_async_copy(rows_v.at[k], acc_sh.at[dst_v.at[b]],
                                  ssem.at[k]).wait()

          pltpu.async_copy(h_hbm.at[src_v.at[b + 2]], rows_v.at[k],
                           gsem.at[k])

    for j in range(RING):
      pltpu.make_async_copy(rows_v.at[j], acc_sh.at[dst_v.at[0]],
                            ssem.at[j]).wait()
    plsc.subcore_barrier()
    pltpu.sync_copy(acc_sh.at[pl.ds(sid * _RPT, _RPT)],
                    out_hbm.at[cid, pl.ds(sid * _RPT, _RPT)])

  return prop


@functools.cache
def _make_deg():

  @functools.partial(
      pl.kernel,
      out_type=jax.ShapeDtypeStruct((_NC, _NP, 16), jnp.float32),
      mesh=_get_mesh(),
      compiler_params=_SC_PARAMS,
      scratch_types=[
          pltpu.VMEM((_NBLK, _B), jnp.int32),
          pltpu.VMEM((_B, 16), jnp.float32),
          pltpu.VMEM_SHARED((_NP, 16), jnp.float32),
      ],
  )
  def deg(e_hbm, ones_hbm, z_hbm, out_hbm, src_v, ones_v, acc_sh):
    cid = lax.axis_index("c")
    sid = lax.axis_index("s")
    wid = cid * _NS + sid
    pltpu.sync_copy(e_hbm.at[0, wid], src_v)
    pltpu.sync_copy(ones_hbm, ones_v)
    pltpu.sync_copy(z_hbm, acc_sh.at[pl.ds(sid * _RPT, _RPT)])
    plsc.subcore_barrier()

    @pl.loop(0, _NBLK)
    def _(b):
      pltpu.sync_copy(ones_v, acc_sh.at[src_v.at[b]], add=True)

    plsc.subcore_barrier()
    pltpu.sync_copy(acc_sh.at[pl.ds(sid * _RPT, _RPT)],
                    out_hbm.at[cid, pl.ds(sid * _RPT, _RPT)])

  return deg


def _tc_matmul1(x, W1):
  def body(x_ref, w_ref, a_ref, c_ref, d_ref):
    xb = x_ref[...]
    w = w_ref[...]
    a_ref[...] = jnp.dot(xb, w[1], preferred_element_type=jnp.float32,
                     precision=jax.lax.Precision.HIGHEST)
    c_ref[...] = 2.0 * jnp.dot(xb, w[2], preferred_element_type=jnp.float32,
                     precision=jax.lax.Precision.HIGHEST)
    d_ref[...] = jnp.dot(xb, w[0] - w[2], preferred_element_type=jnp.float32,
                     precision=jax.lax.Precision.HIGHEST)

  return pl.pallas_call(
      body,
      grid=(_G,),
      in_specs=[
          pl.BlockSpec((_ROWBLK, 128), lambda i: (i, 0)),
          pl.BlockSpec((3, 128, 128), lambda i: (0, 0, 0)),
      ],
      out_specs=[
          pl.BlockSpec((_ROWBLK, 128), lambda i: (i, 0)),
          pl.BlockSpec((_ROWBLK, 128), lambda i: (i, 0)),
          pl.BlockSpec((_ROWBLK, 128), lambda i: (i, 0)),
      ],
      out_shape=[
          jax.ShapeDtypeStruct((_N, 128), jnp.float32),
          jax.ShapeDtypeStruct((_N, 128), jnp.float32),
          jax.ShapeDtypeStruct((_N, 128), jnp.float32),
      ],
  )(x, W1)


def _tc_scale1(c1, degp):
  def body(c_ref, degp_ref, p_ref, dis_ref):
    deg = degp_ref[0, :, 0:1] + degp_ref[1, :, 0:1]
    dis2 = jnp.where(deg > 0.0, lax.rsqrt(jnp.maximum(deg, 1.0)), 0.0)
    p_ref[...] = dis2 * c_ref[...]
    dis_ref[...] = dis2

  return pl.pallas_call(
      body,
      grid=(_G,),
      in_specs=[
          pl.BlockSpec((_ROWBLK, 128), lambda i: (i, 0)),
          pl.BlockSpec((2, _ROWBLK, 16), lambda i: (0, i, 0)),
      ],
      out_specs=[
          pl.BlockSpec((_ROWBLK, 128), lambda i: (i, 0)),
          pl.BlockSpec((_ROWBLK, 1), lambda i: (i, 0)),
      ],
      out_shape=[
          jax.ShapeDtypeStruct((_N, 128), jnp.float32),
          jax.ShapeDtypeStruct((_N, 1), jnp.float32),
      ],
  )(c1, degp)


def _make_tc_combine(F):
  def body(p_ref, a_ref, dis_ref, o_ref):
    dis2 = dis_ref[...]
    s = p_ref[0] + p_ref[1]
    o_ref[...] = dis2 * a_ref[...] - (dis2 * dis2) * s

  def run(Pp, a, dis):
    return pl.pallas_call(
        body,
        grid=(_G,),
        in_specs=[
            pl.BlockSpec((2, _ROWBLK, F), lambda i: (0, i, 0)),
            pl.BlockSpec((_ROWBLK, F), lambda i: (i, 0)),
            pl.BlockSpec((_ROWBLK, 1), lambda i: (i, 0)),
        ],
        out_specs=pl.BlockSpec((_ROWBLK, F), lambda i: (i, 0)),
        out_shape=jax.ShapeDtypeStruct((_N, F), jnp.float32),
    )(Pp, a, dis)

  return run


_combine128 = _make_tc_combine(128)
_combine48 = _make_tc_combine(48)


def _tc_layer2(P2, d1, dis, b1r, W2p):
  def body(p_ref, d1_ref, dis_ref, b1_ref, w_ref, a_ref, p3_ref, d2_ref):
    dis2 = dis_ref[...]
    v = -dis2 * (p_ref[0] + p_ref[1])
    h = jnp.maximum(d1_ref[...] + v + b1_ref[...], 0.0)
    w = w_ref[...]
    a_ref[...] = jnp.dot(h, w[1], preferred_element_type=jnp.float32,
                     precision=jax.lax.Precision.HIGHEST)
    p3_ref[...] = dis2 * (2.0 * jnp.dot(h, w[2],
                                        preferred_element_type=jnp.float32,
                     precision=jax.lax.Precision.HIGHEST))
    d2_ref[...] = jnp.dot(h, w[0] - w[2], preferred_element_type=jnp.float32,
                     precision=jax.lax.Precision.HIGHEST)

  return pl.pallas_call(
      body,
      grid=(_G,),
      in_specs=[
          pl.BlockSpec((2, _ROWBLK, 128), lambda i: (0, i, 0)),
          pl.BlockSpec((_ROWBLK, 128), lambda i: (i, 0)),
          pl.BlockSpec((_ROWBLK, 1), lambda i: (i, 0)),
          pl.BlockSpec((1, 128), lambda i: (0, 0)),
          pl.BlockSpec((3, 128, 48), lambda i: (0, 0, 0)),
      ],
      out_specs=[
          pl.BlockSpec((_ROWBLK, 48), lambda i: (i, 0)),
          pl.BlockSpec((_ROWBLK, 48), lambda i: (i, 0)),
          pl.BlockSpec((_ROWBLK, 48), lambda i: (i, 0)),
      ],
      out_shape=[
          jax.ShapeDtypeStruct((_N, 48), jnp.float32),
          jax.ShapeDtypeStruct((_N, 48), jnp.float32),
          jax.ShapeDtypeStruct((_N, 48), jnp.float32),
      ],
  )(P2, d1, dis, b1r, W2p)


def _tc_final(P4, d2, dis, b2r):
  def body(p_ref, d2_ref, dis_ref, b2_ref, o_ref):
    dis2 = dis_ref[...]
    o_ref[...] = d2_ref[...] - dis2 * (p_ref[0] + p_ref[1]) + b2_ref[...]

  return pl.pallas_call(
      body,
      grid=(_G,),
      in_specs=[
          pl.BlockSpec((2, _ROWBLK, 48), lambda i: (0, i, 0)),
          pl.BlockSpec((_ROWBLK, 48), lambda i: (i, 0)),
          pl.BlockSpec((_ROWBLK, 1), lambda i: (i, 0)),
          pl.BlockSpec((1, 48), lambda i: (0, 0)),
      ],
      out_specs=pl.BlockSpec((_ROWBLK, 48), lambda i: (i, 0)),
      out_shape=jax.ShapeDtypeStruct((_N, 48), jnp.float32),
  )(P4, d2, dis, b2r)


def kernel(x, edge_index, W1, b1, W2, b2):
  e4 = edge_index.reshape(2, _NW, _NBLK, _B)
  e4_128 = edge_index.reshape(2, _NW, _EPT // _PROP_B[128], _PROP_B[128])
  e4_48 = edge_index.reshape(2, _NW, _EPT // _PROP_B[48], _PROP_B[48])
  zeros128 = jnp.zeros((_RPT, 128), jnp.float32)
  zeros48 = jnp.zeros((_RPT, 48), jnp.float32)
  zeros16 = jnp.zeros((_RPT, 16), jnp.float32)
  ones16 = jnp.ones((_B, 16), jnp.float32)
  W2p = jnp.pad(W2, ((0, 0), (0, 0), (0, 8)))
  b1r = b1.reshape(1, 128)
  b2r = jnp.pad(b2, (0, 8)).reshape(1, 48)

  degp = _make_deg()(e4, ones16, zeros16)
  a1, c1, d1 = _tc_matmul1(x, W1)
  p1, dis = _tc_scale1(c1, degp)
  P1 = _make_prop(128)(p1, e4_128, zeros128)
  p2 = _combine128(P1, a1, dis)
  P2 = _make_prop(128)(p2, e4_128, zeros128)
  a2, p3, d2 = _tc_layer2(P2, d1, dis, b1r, W2p)
  P3 = _make_prop(48)(p3, e4_48, zeros48)
  p4 = _combine48(P3, a2, dis)
  P4 = _make_prop(48)(p4, e4_48, zeros48)
  o48 = _tc_final(P4, d2, dis, b2r)
  return o48[:, :40]


# speedup vs baseline: 1.0411x; 1.0411x over previous
import functools

import jax
import jax.numpy as jnp
from jax import lax
from jax.experimental import pallas as pl
from jax.experimental.pallas import tpu as pltpu
from jax.experimental.pallas import tpu_sc as plsc

_N = 10000
_E = 320000
_NC, _NS = 2, 16
_NW = _NC * _NS
_EPT = _E // _NW
_B = 100
_NBLK = _EPT // _B
_PROP_B = {128: 100, 48: 100}
_PROP_RING = {128: 2, 48: 5}
_NP = 10240
_RPT = _NP // _NS

_ROWBLK = 1000
_G = _N // _ROWBLK

@functools.cache
def _get_mesh():
  return plsc.VectorSubcoreMesh(core_axis_name="c", subcore_axis_name="s")


_SC_PARAMS = pltpu.CompilerParams(use_tc_tiling_on_sc=False)


@functools.cache
def _make_prop(F):
  B = _PROP_B[F]
  NBLK = _EPT // B
  RING = _PROP_RING[F]

  @functools.partial(
      pl.kernel,
      out_type=jax.ShapeDtypeStruct((_NC, _NP, F), jnp.float32),
      mesh=_get_mesh(),
      compiler_params=_SC_PARAMS,
      scratch_types=[
          pltpu.VMEM((NBLK, B), jnp.int32),
          pltpu.VMEM((NBLK, B), jnp.int32),
          pltpu.VMEM((RING, B, F), jnp.float32),
          pltpu.VMEM_SHARED((_NP, F), jnp.float32),
          pltpu.SemaphoreType.DMA((RING,)),
          pltpu.SemaphoreType.DMA((RING,)),
      ],
  )
  def prop(h_hbm, e_hbm, z_hbm, out_hbm, src_v, dst_v, rows_v, acc_sh,
           gsem, ssem):
    cid = lax.axis_index("c")
    sid = lax.axis_index("s")
    wid = cid * _NS + sid
    pltpu.sync_copy(e_hbm.at[0, wid], src_v)
    pltpu.sync_copy(e_hbm.at[1, wid], dst_v)
    pltpu.async_copy(h_hbm.at[src_v.at[0]], rows_v.at[0], gsem.at[0])
    pltpu.async_copy(h_hbm.at[src_v.at[1]], rows_v.at[1], gsem.at[1])
    pltpu.sync_copy(z_hbm, acc_sh.at[pl.ds(sid * _RPT, _RPT)])
    plsc.subcore_barrier()

    @pl.loop(0, NBLK // RING)
    def _(i):
      for j in range(RING):
        b = RING * i + j
        k = (j + 2) % RING
        pltpu.make_async_copy(h_hbm.at[src_v.at[b]], rows_v.at[j],
                              gsem.at[j]).wait()
        pltpu.async_copy(rows_v.at[j], acc_sh.at[dst_v.at[b]], ssem.at[j],
                         add=True)

        @pl.when(b + 2 < NBLK)
        def _():
          @pl.when(b >= RING - 2)
          def _():
            pltpu.make_async_copy(rows_v.at[k], acc_sh.at[dst_v.at[b]],
                                  ssem.at[k]).wait()

          pltpu.async_copy(h_hbm.at[src_v.at[b + 2]], rows_v.at[k],
                           gsem.at[k])

    for j in range(RING):
      pltpu.make_async_copy(rows_v.at[j], acc_sh.at[dst_v.at[0]],
                            ssem.at[j]).wait()
    plsc.subcore_barrier()
    pltpu.sync_copy(acc_sh.at[pl.ds(sid * _RPT, _RPT)],
                    out_hbm.at[cid, pl.ds(sid * _RPT, _RPT)])

  return prop


@functools.cache
def _make_deg():

  @functools.partial(
      pl.kernel,
      out_type=jax.ShapeDtypeStruct((_NC, _NP, 16), jnp.float32),
      mesh=_get_mesh(),
      compiler_params=_SC_PARAMS,
      scratch_types=[
          pltpu.VMEM((_NBLK, _B), jnp.int32),
          pltpu.VMEM((_B, 16), jnp.float32),
          pltpu.VMEM_SHARED((_NP, 16), jnp.float32),
      ],
  )
  def deg(e_hbm, ones_hbm, z_hbm, out_hbm, src_v, ones_v, acc_sh):
    cid = lax.axis_index("c")
    sid = lax.axis_index("s")
    wid = cid * _NS + sid
    pltpu.sync_copy(e_hbm.at[0, wid], src_v)
    pltpu.sync_copy(ones_hbm, ones_v)
    pltpu.sync_copy(z_hbm, acc_sh.at[pl.ds(sid * _RPT, _RPT)])
    plsc.subcore_barrier()

    @pl.loop(0, _NBLK)
    def _(b):
      pltpu.sync_copy(ones_v, acc_sh.at[src_v.at[b]], add=True)

    plsc.subcore_barrier()
    pltpu.sync_copy(acc_sh.at[pl.ds(sid * _RPT, _RPT)],
                    out_hbm.at[cid, pl.ds(sid * _RPT, _RPT)])

  return deg


def _tc_matmul1(x, W1):
  def body(x_ref, w_ref, a_ref, c_ref, d_ref):
    xb = x_ref[...]
    w = w_ref[...]
    a_ref[...] = jnp.dot(xb, w[1], preferred_element_type=jnp.float32)
    c_ref[...] = 2.0 * jnp.dot(xb, w[2], preferred_element_type=jnp.float32)
    d_ref[...] = jnp.dot(xb, w[0] - w[2], preferred_element_type=jnp.float32)

  return pl.pallas_call(
      body,
      grid=(_G,),
      in_specs=[
          pl.BlockSpec((_ROWBLK, 128), lambda i: (i, 0)),
          pl.BlockSpec((3, 128, 128), lambda i: (0, 0, 0)),
      ],
      out_specs=[
          pl.BlockSpec((_ROWBLK, 128), lambda i: (i, 0)),
          pl.BlockSpec((_ROWBLK, 128), lambda i: (i, 0)),
          pl.BlockSpec((_ROWBLK, 128), lambda i: (i, 0)),
      ],
      out_shape=[
          jax.ShapeDtypeStruct((_N, 128), jnp.float32),
          jax.ShapeDtypeStruct((_N, 128), jnp.float32),
          jax.ShapeDtypeStruct((_N, 128), jnp.float32),
      ],
  )(x, W1)


def _tc_scale1(c1, degp):
  def body(c_ref, degp_ref, p_ref, dis_ref):
    deg = degp_ref[0, :, 0:1] + degp_ref[1, :, 0:1]
    dis2 = jnp.where(deg > 0.0, lax.rsqrt(jnp.maximum(deg, 1.0)), 0.0)
    p_ref[...] = dis2 * c_ref[...]
    dis_ref[...] = dis2

  return pl.pallas_call(
      body,
      grid=(_G,),
      in_specs=[
          pl.BlockSpec((_ROWBLK, 128), lambda i: (i, 0)),
          pl.BlockSpec((2, _ROWBLK, 16), lambda i: (0, i, 0)),
      ],
      out_specs=[
          pl.BlockSpec((_ROWBLK, 128), lambda i: (i, 0)),
          pl.BlockSpec((_ROWBLK, 1), lambda i: (i, 0)),
      ],
      out_shape=[
          jax.ShapeDtypeStruct((_N, 128), jnp.float32),
          jax.ShapeDtypeStruct((_N, 1), jnp.float32),
      ],
  )(c1, degp)


def _make_tc_combine(F):
  def body(p_ref, a_ref, dis_ref, o_ref):
    dis2 = dis_ref[...]
    s = p_ref[0] + p_ref[1]
    o_ref[...] = dis2 * a_ref[...] - (dis2 * dis2) * s

  def run(Pp, a, dis):
    return pl.pallas_call(
        body,
        grid=(_G,),
        in_specs=[
            pl.BlockSpec((2, _ROWBLK, F), lambda i: (0, i, 0)),
            pl.BlockSpec((_ROWBLK, F), lambda i: (i, 0)),
            pl.BlockSpec((_ROWBLK, 1), lambda i: (i, 0)),
        ],
        out_specs=pl.BlockSpec((_ROWBLK, F), lambda i: (i, 0)),
        out_shape=jax.ShapeDtypeStruct((_N, F), jnp.float32),
    )(Pp, a, dis)

  return run


_combine128 = _make_tc_combine(128)
_combine48 = _make_tc_combine(48)


def _tc_layer2(P2, d1, dis, b1r, W2p):
  def body(p_ref, d1_ref, dis_ref, b1_ref, w_ref, a_ref, p3_ref, d2_ref):
    dis2 = dis_ref[...]
    v = -dis2 * (p_ref[0] + p_ref[1])
    h = jnp.maximum(d1_ref[...] + v + b1_ref[...], 0.0)
    w = w_ref[...]
    a_ref[...] = jnp.dot(h, w[1], preferred_element_type=jnp.float32)
    p3_ref[...] = dis2 * (2.0 * jnp.dot(h, w[2],
                                        preferred_element_type=jnp.float32))
    d2_ref[...] = jnp.dot(h, w[0] - w[2], preferred_element_type=jnp.float32)

  return pl.pallas_call(
      body,
      grid=(_G,),
      in_specs=[
          pl.BlockSpec((2, _ROWBLK, 128), lambda i: (0, i, 0)),
          pl.BlockSpec((_ROWBLK, 128), lambda i: (i, 0)),
          pl.BlockSpec((_ROWBLK, 1), lambda i: (i, 0)),
          pl.BlockSpec((1, 128), lambda i: (0, 0)),
          pl.BlockSpec((3, 128, 48), lambda i: (0, 0, 0)),
      ],
      out_specs=[
          pl.BlockSpec((_ROWBLK, 48), lambda i: (i, 0)),
          pl.BlockSpec((_ROWBLK, 48), lambda i: (i, 0)),
          pl.BlockSpec((_ROWBLK, 48), lambda i: (i, 0)),
      ],
      out_shape=[
          jax.ShapeDtypeStruct((_N, 48), jnp.float32),
          jax.ShapeDtypeStruct((_N, 48), jnp.float32),
          jax.ShapeDtypeStruct((_N, 48), jnp.float32),
      ],
  )(P2, d1, dis, b1r, W2p)


def _tc_final(P4, d2, dis, b2r):
  def body(p_ref, d2_ref, dis_ref, b2_ref, o_ref):
    dis2 = dis_ref[...]
    o_ref[...] = d2_ref[...] - dis2 * (p_ref[0] + p_ref[1]) + b2_ref[...]

  return pl.pallas_call(
      body,
      grid=(_G,),
      in_specs=[
          pl.BlockSpec((2, _ROWBLK, 48), lambda i: (0, i, 0)),
          pl.BlockSpec((_ROWBLK, 48), lambda i: (i, 0)),
          pl.BlockSpec((_ROWBLK, 1), lambda i: (i, 0)),
          pl.BlockSpec((1, 48), lambda i: (0, 0)),
      ],
      out_specs=pl.BlockSpec((_ROWBLK, 48), lambda i: (i, 0)),
      out_shape=jax.ShapeDtypeStruct((_N, 48), jnp.float32),
  )(P4, d2, dis, b2r)


def kernel(x, edge_index, W1, b1, W2, b2):
  e4 = edge_index.reshape(2, _NW, _NBLK, _B)
  e4_128 = edge_index.reshape(2, _NW, _EPT // _PROP_B[128], _PROP_B[128])
  e4_48 = edge_index.reshape(2, _NW, _EPT // _PROP_B[48], _PROP_B[48])
  zeros128 = jnp.zeros((_RPT, 128), jnp.float32)
  zeros48 = jnp.zeros((_RPT, 48), jnp.float32)
  zeros16 = jnp.zeros((_RPT, 16), jnp.float32)
  ones16 = jnp.ones((_B, 16), jnp.float32)
  W2p = jnp.pad(W2, ((0, 0), (0, 0), (0, 8)))
  b1r = b1.reshape(1, 128)
  b2r = jnp.pad(b2, (0, 8)).reshape(1, 48)

  degp = _make_deg()(e4, ones16, zeros16)
  a1, c1, d1 = _tc_matmul1(x, W1)
  p1, dis = _tc_scale1(c1, degp)
  P1 = _make_prop(128)(p1, e4_128, zeros128)
  p2 = _combine128(P1, a1, dis)
  P2 = _make_prop(128)(p2, e4_128, zeros128)
  a2, p3, d2 = _tc_layer2(P2, d1, dis, b1r, W2p)
  P3 = _make_prop(48)(p3, e4_48, zeros48)
  p4 = _combine48(P3, a2, dis)
  P4 = _make_prop(48)(p4, e4_48, zeros48)
  o48 = _tc_final(P4, d2, dis, b2r)
  return o48[:, :40]


# speedup vs baseline: 1.0507x; 1.0092x over previous
import functools

import jax
import jax.numpy as jnp
from jax import lax
from jax.experimental import pallas as pl
from jax.experimental.pallas import tpu as pltpu
from jax.experimental.pallas import tpu_sc as plsc

_N = 10000
_E = 320000
_NC, _NS = 2, 16
_NW = _NC * _NS
_EPT = _E // _NW
_B = 100
_NBLK = _EPT // _B
_PROP_B = {128: 100, 48: 100}
_PROP_RING = {128: 2, 48: 5}
_NP = 10240
_RPT = _NP // _NS

_ROWBLK = 1000
_G = _N // _ROWBLK

@functools.cache
def _get_mesh():
  return plsc.VectorSubcoreMesh(core_axis_name="c", subcore_axis_name="s")


_SC_PARAMS = pltpu.CompilerParams(use_tc_tiling_on_sc=False)


@functools.cache
def _make_prop(F):
  B = _PROP_B[F]
  NBLK = _EPT // B
  RING = _PROP_RING[F]

  @functools.partial(
      pl.kernel,
      out_type=jax.ShapeDtypeStruct((_NC, _NP, F), jnp.float32),
      mesh=_get_mesh(),
      compiler_params=_SC_PARAMS,
      scratch_types=[
          pltpu.VMEM((NBLK, B), jnp.int32),
          pltpu.VMEM((NBLK, B), jnp.int32),
          pltpu.VMEM((RING, B, F), jnp.float32),
          pltpu.VMEM_SHARED((_NP, F), jnp.float32),
          pltpu.SemaphoreType.DMA((RING,)),
          pltpu.SemaphoreType.DMA((RING,)),
      ],
  )
  def prop(h_hbm, e_hbm, z_hbm, out_hbm, src_v, dst_v, rows_v, acc_sh,
           gsem, ssem):
    cid = lax.axis_index("c")
    sid = lax.axis_index("s")
    wid = cid * _NS + sid
    pltpu.sync_copy(e_hbm.at[0, wid], src_v)
    pltpu.sync_copy(e_hbm.at[1, wid], dst_v)
    pltpu.async_copy(h_hbm.at[src_v.at[0]], rows_v.at[0], gsem.at[0])
    pltpu.async_copy(h_hbm.at[src_v.at[1]], rows_v.at[1], gsem.at[1])
    pltpu.sync_copy(z_hbm, acc_sh.at[pl.ds(sid * _RPT, _RPT)])
    plsc.subcore_barrier()

    @pl.loop(0, NBLK // RING)
    def _(i):
      for j in range(RING):
        b = RING * i + j
        k = (j + 2) % RING
        pltpu.make_async_copy(h_hbm.at[src_v.at[b]], rows_v.at[j],
                              gsem.at[j]).wait()
        pltpu.async_copy(rows_v.at[j], acc_sh.at[dst_v.at[b]], ssem.at[j],
                         add=True)

        @pl.when(b + 2 < NBLK)
        def _():
          @pl.when(b >= RING - 2)
          def _():
            pltpu.make_async_copy(rows_v.at[k], acc_sh.at[dst_v.at[b]],
                                  ssem.at[k]).wait()

          pltpu.async_copy(h_hbm.at[src_v.at[b + 2]], rows_v.at[k],
                           gsem.at[k])

    for j in range(RING):
      pltpu.make_async_copy(rows_v.at[j], acc_sh.at[dst_v.at[0]],
                            ssem.at[j]).wait()
    plsc.subcore_barrier()
    pltpu.sync_copy(acc_sh.at[pl.ds(sid * _RPT, _RPT)],
                    out_hbm.at[cid, pl.ds(sid * _RPT, _RPT)])

  return prop


@functools.cache
def _make_deg():

  @functools.partial(
      pl.kernel,
      out_type=jax.ShapeDtypeStruct((_NC, _NP, 16), jnp.float32),
      mesh=_get_mesh(),
      compiler_params=_SC_PARAMS,
      scratch_types=[
          pltpu.VMEM((_NBLK, _B), jnp.int32),
          pltpu.VMEM((_B, 16), jnp.float32),
          pltpu.VMEM_SHARED((_NP, 16), jnp.float32),
          pltpu.SemaphoreType.DMA,
      ],
  )
  def deg(e_hbm, ones_hbm, z_hbm, out_hbm, src_v, ones_v, acc_sh, dsem):
    cid = lax.axis_index("c")
    sid = lax.axis_index("s")
    wid = cid * _NS + sid
    pltpu.sync_copy(e_hbm.at[0, wid], src_v)
    pltpu.sync_copy(ones_hbm, ones_v)
    pltpu.sync_copy(z_hbm, acc_sh.at[pl.ds(sid * _RPT, _RPT)])
    plsc.subcore_barrier()

    @pl.loop(0, _NBLK)
    def _(b):
      pltpu.async_copy(ones_v, acc_sh.at[src_v.at[b]], dsem, add=True)

    @pl.loop(0, _NBLK)
    def _(b):
      pltpu.make_async_copy(ones_v, acc_sh.at[src_v.at[0]], dsem).wait()

    plsc.subcore_barrier()
    pltpu.sync_copy(acc_sh.at[pl.ds(sid * _RPT, _RPT)],
                    out_hbm.at[cid, pl.ds(sid * _RPT, _RPT)])

  return deg


def _tc_matmul1(x, W1):
  def body(x_ref, w_ref, a_ref, c_ref, d_ref):
    xb = x_ref[...]
    w = w_ref[...]
    a_ref[...] = jnp.dot(xb, w[1], preferred_element_type=jnp.float32)
    c_ref[...] = 2.0 * jnp.dot(xb, w[2], preferred_element_type=jnp.float32)
    d_ref[...] = jnp.dot(xb, w[0] - w[2], preferred_element_type=jnp.float32)

  return pl.pallas_call(
      body,
      grid=(_G,),
      in_specs=[
          pl.BlockSpec((_ROWBLK, 128), lambda i: (i, 0)),
          pl.BlockSpec((3, 128, 128), lambda i: (0, 0, 0)),
      ],
      out_specs=[
          pl.BlockSpec((_ROWBLK, 128), lambda i: (i, 0)),
          pl.BlockSpec((_ROWBLK, 128), lambda i: (i, 0)),
          pl.BlockSpec((_ROWBLK, 128), lambda i: (i, 0)),
      ],
      out_shape=[
          jax.ShapeDtypeStruct((_N, 128), jnp.float32),
          jax.ShapeDtypeStruct((_N, 128), jnp.float32),
          jax.ShapeDtypeStruct((_N, 128), jnp.float32),
      ],
  )(x, W1)


def _tc_scale1(c1, degp):
  def body(c_ref, degp_ref, p_ref, dis_ref):
    deg = degp_ref[0, :, 0:1] + degp_ref[1, :, 0:1]
    dis2 = jnp.where(deg > 0.0, lax.rsqrt(jnp.maximum(deg, 1.0)), 0.0)
    p_ref[...] = dis2 * c_ref[...]
    dis_ref[...] = dis2

  return pl.pallas_call(
      body,
      grid=(_G,),
      in_specs=[
          pl.BlockSpec((_ROWBLK, 128), lambda i: (i, 0)),
          pl.BlockSpec((2, _ROWBLK, 16), lambda i: (0, i, 0)),
      ],
      out_specs=[
          pl.BlockSpec((_ROWBLK, 128), lambda i: (i, 0)),
          pl.BlockSpec((_ROWBLK, 1), lambda i: (i, 0)),
      ],
      out_shape=[
          jax.ShapeDtypeStruct((_N, 128), jnp.float32),
          jax.ShapeDtypeStruct((_N, 1), jnp.float32),
      ],
  )(c1, degp)


def _make_tc_combine(F):
  def body(p_ref, a_ref, dis_ref, o_ref):
    dis2 = dis_ref[...]
    s = p_ref[0] + p_ref[1]
    o_ref[...] = dis2 * a_ref[...] - (dis2 * dis2) * s

  def run(Pp, a, dis):
    return pl.pallas_call(
        body,
        grid=(_G,),
        in_specs=[
            pl.BlockSpec((2, _ROWBLK, F), lambda i: (0, i, 0)),
            pl.BlockSpec((_ROWBLK, F), lambda i: (i, 0)),
            pl.BlockSpec((_ROWBLK, 1), lambda i: (i, 0)),
        ],
        out_specs=pl.BlockSpec((_ROWBLK, F), lambda i: (i, 0)),
        out_shape=jax.ShapeDtypeStruct((_N, F), jnp.float32),
    )(Pp, a, dis)

  return run


_combine128 = _make_tc_combine(128)
_combine48 = _make_tc_combine(48)


def _tc_layer2(P2, d1, dis, b1r, W2p):
  def body(p_ref, d1_ref, dis_ref, b1_ref, w_ref, a_ref, p3_ref, d2_ref):
    dis2 = dis_ref[...]
    v = -dis2 * (p_ref[0] + p_ref[1])
    h = jnp.maximum(d1_ref[...] + v + b1_ref[...], 0.0)
    w = w_ref[...]
    a_ref[...] = jnp.dot(h, w[1], preferred_element_type=jnp.float32)
    p3_ref[...] = dis2 * (2.0 * jnp.dot(h, w[2],
                                        preferred_element_type=jnp.float32))
    d2_ref[...] = jnp.dot(h, w[0] - w[2], preferred_element_type=jnp.float32)

  return pl.pallas_call(
      body,
      grid=(_G,),
      in_specs=[
          pl.BlockSpec((2, _ROWBLK, 128), lambda i: (0, i, 0)),
          pl.BlockSpec((_ROWBLK, 128), lambda i: (i, 0)),
          pl.BlockSpec((_ROWBLK, 1), lambda i: (i, 0)),
          pl.BlockSpec((1, 128), lambda i: (0, 0)),
          pl.BlockSpec((3, 128, 48), lambda i: (0, 0, 0)),
      ],
      out_specs=[
          pl.BlockSpec((_ROWBLK, 48), lambda i: (i, 0)),
          pl.BlockSpec((_ROWBLK, 48), lambda i: (i, 0)),
          pl.BlockSpec((_ROWBLK, 48), lambda i: (i, 0)),
      ],
      out_shape=[
          jax.ShapeDtypeStruct((_N, 48), jnp.float32),
          jax.ShapeDtypeStruct((_N, 48), jnp.float32),
          jax.ShapeDtypeStruct((_N, 48), jnp.float32),
      ],
  )(P2, d1, dis, b1r, W2p)


def _tc_final(P4, d2, dis, b2r):
  def body(p_ref, d2_ref, dis_ref, b2_ref, o_ref):
    dis2 = dis_ref[...]
    o_ref[...] = d2_ref[...] - dis2 * (p_ref[0] + p_ref[1]) + b2_ref[...]

  return pl.pallas_call(
      body,
      grid=(_G,),
      in_specs=[
          pl.BlockSpec((2, _ROWBLK, 48), lambda i: (0, i, 0)),
          pl.BlockSpec((_ROWBLK, 48), lambda i: (i, 0)),
          pl.BlockSpec((_ROWBLK, 1), lambda i: (i, 0)),
          pl.BlockSpec((1, 48), lambda i: (0, 0)),
      ],
      out_specs=pl.BlockSpec((_ROWBLK, 48), lambda i: (i, 0)),
      out_shape=jax.ShapeDtypeStruct((_N, 48), jnp.float32),
  )(P4, d2, dis, b2r)


def kernel(x, edge_index, W1, b1, W2, b2):
  e4 = edge_index.reshape(2, _NW, _NBLK, _B)
  e4_128 = edge_index.reshape(2, _NW, _EPT // _PROP_B[128], _PROP_B[128])
  e4_48 = edge_index.reshape(2, _NW, _EPT // _PROP_B[48], _PROP_B[48])
  zeros128 = jnp.zeros((_RPT, 128), jnp.float32)
  zeros48 = jnp.zeros((_RPT, 48), jnp.float32)
  zeros16 = jnp.zeros((_RPT, 16), jnp.float32)
  ones16 = jnp.ones((_B, 16), jnp.float32)
  W2p = jnp.pad(W2, ((0, 0), (0, 0), (0, 8)))
  b1r = b1.reshape(1, 128)
  b2r = jnp.pad(b2, (0, 8)).reshape(1, 48)

  degp = _make_deg()(e4, ones16, zeros16)
  a1, c1, d1 = _tc_matmul1(x, W1)
  p1, dis = _tc_scale1(c1, degp)
  P1 = _make_prop(128)(p1, e4_128, zeros128)
  p2 = _combine128(P1, a1, dis)
  P2 = _make_prop(128)(p2, e4_128, zeros128)
  a2, p3, d2 = _tc_layer2(P2, d1, dis, b1r, W2p)
  P3 = _make_prop(48)(p3, e4_48, zeros48)
  p4 = _combine48(P3, a2, dis)
  P4 = _make_prop(48)(p4, e4_48, zeros48)
  o48 = _tc_final(P4, d2, dis, b2r)
  return o48[:, :40]
